# Initial kernel scaffold; baseline (speedup 1.0000x reference)
import jax
import jax.numpy as jnp
from jax.experimental import pallas as pl


def kernel(ids, table):
    raise NotImplementedError("write your pallas kernel here")


# speedup vs baseline: 2.9580x; 2.9580x over previous
import functools

import jax
import jax.numpy as jnp
from jax import lax
from jax.experimental import pallas as pl
from jax.experimental.pallas import tpu as pltpu
from jax.experimental.pallas import tpu_sc as plsc

NC = 2
NS = 16
NW = NC * NS

EMB_DIM = 128
CHUNK = 128


def _build_gather(n_rows: int, emb_dim: int):
    assert n_rows % (NW * CHUNK) == 0
    rows_per_w = n_rows // NW
    n_chunks = rows_per_w // CHUNK

    mesh = plsc.VectorSubcoreMesh(core_axis_name="c", subcore_axis_name="s")

    @functools.partial(
        pl.kernel,
        out_type=jax.ShapeDtypeStruct((n_rows, emb_dim), jnp.float32),
        mesh=mesh,
        scratch_types=[
            pltpu.VMEM((n_chunks, CHUNK), jnp.int32),
            pltpu.VMEM((CHUNK, emb_dim), jnp.float32),
            pltpu.SemaphoreType.DMA,
        ],
    )
    def gather_kernel(ids_hbm, table_hbm, out_hbm, idx_v, buf, sem):
        w = lax.axis_index("s") * NC + lax.axis_index("c")
        pltpu.sync_copy(ids_hbm.at[w], idx_v)

        def step(j, carry):
            pltpu.async_copy(table_hbm.at[idx_v.at[j]], buf, sem).wait()
            pltpu.sync_copy(buf, out_hbm.at[pl.ds(w * rows_per_w + j * CHUNK, CHUNK)])
            return carry

        lax.fori_loop(0, n_chunks, step, 0)

    return gather_kernel


def kernel(ids, table):
    b, l = ids.shape
    n_rows = b * l
    idx2d = ids.reshape(NW, n_rows // (NW * CHUNK), CHUNK).astype(jnp.int32)
    out = _build_gather(n_rows, table.shape[1])(idx2d, table)
    return out.reshape(b, l, table.shape[1])


# speedup vs baseline: 3.3189x; 1.1220x over previous
import functools

import jax
import jax.numpy as jnp
from jax import lax
from jax.experimental import pallas as pl
from jax.experimental.pallas import tpu as pltpu
from jax.experimental.pallas import tpu_sc as plsc

NC = 2
NS = 16
NW = NC * NS

EMB_DIM = 128
CHUNK = 128


def _build_gather(n_rows: int, emb_dim: int):
    assert n_rows % (NW * CHUNK) == 0
    rows_per_w = n_rows // NW
    n_chunks = rows_per_w // CHUNK

    mesh = plsc.VectorSubcoreMesh(core_axis_name="c", subcore_axis_name="s")

    assert n_chunks % 2 == 0
    n_rounds = n_chunks // 2

    @functools.partial(
        pl.kernel,
        out_type=jax.ShapeDtypeStruct((n_rows, emb_dim), jnp.float32),
        mesh=mesh,
        scratch_types=[
            pltpu.VMEM((n_chunks, CHUNK), jnp.int32),
            pltpu.VMEM((CHUNK, emb_dim), jnp.float32),
            pltpu.VMEM((CHUNK, emb_dim), jnp.float32),
            pltpu.SemaphoreType.DMA,
            pltpu.SemaphoreType.DMA,
        ],
    )
    def gather_kernel(ids_hbm, table_hbm, out_hbm, idx_v, buf0, buf1, sem0, sem1):
        w = lax.axis_index("s") * NC + lax.axis_index("c")
        pltpu.sync_copy(ids_hbm.at[w], idx_v)

        def fire(j, buf, sem):
            pltpu.async_copy(table_hbm.at[idx_v.at[j]], buf, sem)

        def drain(j, buf, sem):
            pltpu.make_async_copy(table_hbm.at[idx_v.at[j]], buf, sem).wait()

        def write(j, buf):
            pltpu.sync_copy(buf, out_hbm.at[pl.ds(w * rows_per_w + j * CHUNK, CHUNK)])

        fire(0, buf0, sem0)

        def step(k, carry):
            j0 = 2 * k
            j1 = j0 + 1
            fire(j1, buf1, sem1)
            drain(j0, buf0, sem0)
            write(j0, buf0)

            @pl.when(k + 1 < n_rounds)
            def _():
                fire(j0 + 2, buf0, sem0)

            drain(j1, buf1, sem1)
            write(j1, buf1)
            return carry

        lax.fori_loop(0, n_rounds, step, 0)

    return gather_kernel


def kernel(ids, table):
    b, l = ids.shape
    n_rows = b * l
    idx2d = ids.reshape(NW, n_rows // (NW * CHUNK), CHUNK).astype(jnp.int32)
    out = _build_gather(n_rows, table.shape[1])(idx2d, table)
    return out.reshape(b, l, table.shape[1])


# speedup vs baseline: 3.3384x; 1.0059x over previous
import functools

import jax
import jax.numpy as jnp
from jax import lax
from jax.experimental import pallas as pl
from jax.experimental.pallas import tpu as pltpu
from jax.experimental.pallas import tpu_sc as plsc

NC = 2
NS = 16
NW = NC * NS

EMB_DIM = 128
CHUNK = 128


def _build_gather(n_rows: int, emb_dim: int):
    assert n_rows % (NW * CHUNK) == 0
    rows_per_w = n_rows // NW
    n_chunks = rows_per_w // CHUNK

    mesh = plsc.VectorSubcoreMesh(core_axis_name="c", subcore_axis_name="s")

    nbuf = 5
    assert n_chunks % nbuf == 0
    ahead = 2

    @functools.partial(
        pl.kernel,
        out_type=jax.ShapeDtypeStruct((n_rows, emb_dim), jnp.float32),
        mesh=mesh,
        scratch_types=[
            pltpu.VMEM((n_chunks, CHUNK), jnp.int32),
            [pltpu.VMEM((CHUNK, emb_dim), jnp.float32) for _ in range(nbuf)],
            [pltpu.SemaphoreType.DMA for _ in range(nbuf)],
            [pltpu.SemaphoreType.DMA for _ in range(nbuf)],
        ],
    )
    def gather_kernel(ids_hbm, table_hbm, out_hbm, idx_v, bufs, gsems, wsems):
        w = lax.axis_index("s") * NC + lax.axis_index("c")
        pltpu.sync_copy(ids_hbm.at[w], idx_v)

        def fire_gather(j, b):
            pltpu.async_copy(table_hbm.at[idx_v.at[j]], bufs[b], gsems[b])

        def wait_gather(j, b):
            pltpu.make_async_copy(table_hbm.at[idx_v.at[j]], bufs[b], gsems[b]).wait()

        def out_slice(j):
            return out_hbm.at[pl.ds(w * rows_per_w + j * CHUNK, CHUNK)]

        def fire_write(j, b):
            pltpu.async_copy(bufs[b], out_slice(j), wsems[b])

        def wait_write(j, b):
            pltpu.make_async_copy(bufs[b], out_slice(j), wsems[b]).wait()

        for j in range(ahead):
            fire_gather(j, j % nbuf)

        def step(k, carry):
            for b in range(nbuf):
                j = k * nbuf + b
                wait_gather(j, b)
                fire_write(j, b)
                bn = (b + ahead) % nbuf

                @pl.when(j + ahead - nbuf >= 0)
                def _():
                    wait_write(j + ahead - nbuf, bn)

                @pl.when(j + ahead < n_chunks)
                def _():
                    fire_gather(j + ahead, bn)
            return carry

        lax.fori_loop(0, n_chunks // nbuf, step, 0)

        for j in range(n_chunks - nbuf + ahead, n_chunks):
            wait_write(j, j % nbuf)

    return gather_kernel


def kernel(ids, table):
    b, l = ids.shape
    n_rows = b * l
    idx2d = ids.reshape(NW, n_rows // (NW * CHUNK), CHUNK).astype(jnp.int32)
    out = _build_gather(n_rows, table.shape[1])(idx2d, table)
    return out.reshape(b, l, table.shape[1])


# speedup vs baseline: 5.4942x; 1.6458x over previous
import functools

import jax
import jax.numpy as jnp
from jax import lax
from jax.experimental import pallas as pl
from jax.experimental.pallas import tpu as pltpu
from jax.experimental.pallas import tpu_sc as plsc

NC = 2
NS = 16
NW = NC * NS


def _build_gather(batch: int, hist: int, emb_dim: int):
    assert batch % NW == 0
    bpw = batch // NW
    nbuf = 4
    ahead = 2
    assert bpw % nbuf == 0

    mesh = plsc.VectorSubcoreMesh(core_axis_name="c", subcore_axis_name="s")

    @functools.partial(
        pl.kernel,
        out_type=jax.ShapeDtypeStruct((batch, hist, emb_dim), jnp.float32),
        mesh=mesh,
        compiler_params=pltpu.CompilerParams(use_tc_tiling_on_sc=True),
        scratch_types=[
            pltpu.VMEM((bpw, hist), jnp.int32),
            [pltpu.VMEM((hist, emb_dim), jnp.float32) for _ in range(nbuf)],
            [pltpu.SemaphoreType.DMA for _ in range(nbuf)],
            [pltpu.SemaphoreType.DMA for _ in range(nbuf)],
        ],
    )
    def gather_kernel(ids_hbm, table_hbm, out_hbm, idx_v, bufs, gsems, wsems):
        w = lax.axis_index("s") * NC + lax.axis_index("c")
        pltpu.sync_copy(ids_hbm.at[pl.ds(w * bpw, bpw)], idx_v)

        def fire_gather(i, b):
            pltpu.async_copy(table_hbm.at[idx_v.at[i]], bufs[b], gsems[b])

        def wait_gather(i, b):
            pltpu.make_async_copy(table_hbm.at[idx_v.at[i]], bufs[b], gsems[b]).wait()

        def fire_write(i, b):
            pltpu.async_copy(bufs[b], out_hbm.at[w * bpw + i], wsems[b])

        def wait_write(i, b):
            pltpu.make_async_copy(bufs[b], out_hbm.at[w * bpw + i], wsems[b]).wait()

        for i in range(ahead):
            fire_gather(i, i % nbuf)

        def step(k, carry):
            for b in range(nbuf):
                i = k * nbuf + b
                wait_gather(i, b)
                fire_write(i, b)
                bn = (b + ahead) % nbuf

                @pl.when(i + ahead - nbuf >= 0)
                def _():
                    wait_write(i + ahead - nbuf, bn)

                @pl.when(i + ahead < bpw)
                def _():
                    fire_gather(i + ahead, bn)
            return carry

        lax.fori_loop(0, bpw // nbuf, step, 0)

        for i in range(bpw - nbuf + ahead, bpw):
            wait_write(i, i % nbuf)

    return gather_kernel


def kernel(ids, table):
    b, l = ids.shape
    out = _build_gather(b, l, table.shape[1])(ids.astype(jnp.int32), table)
    return out


# speedup vs baseline: 5.9350x; 1.0802x over previous
import functools

import jax
import jax.numpy as jnp
from jax import lax
from jax.experimental import pallas as pl
from jax.experimental.pallas import tpu as pltpu
from jax.experimental.pallas import tpu_sc as plsc

NC = 2
NS = 16
NW = NC * NS


def _build_gather(batch: int, hist: int, emb_dim: int):
    assert batch % NW == 0
    bpw = batch // NW
    rows = 4
    nbuf = 4
    ahead = 2
    assert bpw % (rows * nbuf) == 0
    n_steps = bpw // rows

    mesh = plsc.VectorSubcoreMesh(core_axis_name="c", subcore_axis_name="s")

    @functools.partial(
        pl.kernel,
        out_type=jax.ShapeDtypeStruct((batch, hist, emb_dim), jnp.float32),
        mesh=mesh,
        compiler_params=pltpu.CompilerParams(use_tc_tiling_on_sc=True),
        scratch_types=[
            pltpu.VMEM((bpw, hist), jnp.int32),
            [pltpu.VMEM((rows, hist, emb_dim), jnp.float32) for _ in range(nbuf)],
            [pltpu.SemaphoreType.DMA for _ in range(nbuf)],
            [pltpu.SemaphoreType.DMA for _ in range(nbuf)],
        ],
    )
    def gather_kernel(ids_hbm, table_hbm, out_hbm, idx_v, bufs, gsems, wsems):
        w = lax.axis_index("s") * NC + lax.axis_index("c")
        pltpu.sync_copy(ids_hbm.at[pl.ds(w * bpw, bpw)], idx_v)

        def fire_gather(i, b):
            for r in range(rows):
                pltpu.async_copy(
                    table_hbm.at[idx_v.at[i * rows + r]], bufs[b].at[r], gsems[b])

        def wait_gather(i, b):
            for r in range(rows):
                pltpu.make_async_copy(
                    table_hbm.at[idx_v.at[i * rows + r]], bufs[b].at[r], gsems[b]).wait()

        def out_slice(i):
            return out_hbm.at[pl.ds(w * bpw + i * rows, rows)]

        def fire_write(i, b):
            pltpu.async_copy(bufs[b], out_slice(i), wsems[b])

        def wait_write(i, b):
            pltpu.make_async_copy(bufs[b], out_slice(i), wsems[b]).wait()

        for i in range(ahead):
            fire_gather(i, i % nbuf)

        def step(k, carry):
            for b in range(nbuf):
                i = k * nbuf + b
                wait_gather(i, b)
                fire_write(i, b)
                bn = (b + ahead) % nbuf

                @pl.when(i + ahead - nbuf >= 0)
                def _():
                    wait_write(i + ahead - nbuf, bn)

                @pl.when(i + ahead < n_steps)
                def _():
                    fire_gather(i + ahead, bn)
            return carry

        lax.fori_loop(0, n_steps // nbuf, step, 0)

        for i in range(n_steps - nbuf + ahead, n_steps):
            wait_write(i, i % nbuf)

    return gather_kernel


def kernel(ids, table):
    b, l = ids.shape
    out = _build_gather(b, l, table.shape[1])(ids.astype(jnp.int32), table)
    return out


# speedup vs baseline: 10.5943x; 1.7851x over previous
import functools

import jax
import jax.numpy as jnp
from jax import lax
from jax.experimental import pallas as pl
from jax.experimental.pallas import tpu as pltpu
from jax.experimental.pallas import tpu_sc as plsc

NC = 2
NS = 16
NW = NC * NS


def _build_gather(batch: int, hist: int, emb_dim: int):
    assert batch % NW == 0
    bpw = batch // NW
    nbuf = 5
    ahead = 2
    assert hist % nbuf == 0

    mesh = plsc.VectorSubcoreMesh(core_axis_name="c", subcore_axis_name="s")

    @functools.partial(
        pl.kernel,
        out_type=jax.ShapeDtypeStruct((hist, batch, emb_dim), jnp.float32),
        mesh=mesh,
        compiler_params=pltpu.CompilerParams(use_tc_tiling_on_sc=True),
        scratch_types=[
            pltpu.VMEM((hist, bpw), jnp.int32),
            [pltpu.VMEM((bpw, emb_dim), jnp.float32) for _ in range(nbuf)],
            [pltpu.SemaphoreType.DMA for _ in range(nbuf)],
            [pltpu.SemaphoreType.DMA for _ in range(nbuf)],
        ],
    )
    def gather_kernel(ids_hbm, table_hbm, out_hbm, idx_v, bufs, gsems, wsems):
        w = lax.axis_index("s") * NC + lax.axis_index("c")
        pltpu.sync_copy(ids_hbm.at[:, pl.ds(w * bpw, bpw)], idx_v)

        def fire_gather(l, b):
            pltpu.async_copy(table_hbm.at[idx_v.at[l]], bufs[b], gsems[b])

        def wait_gather(l, b):
            pltpu.make_async_copy(table_hbm.at[idx_v.at[l]], bufs[b], gsems[b]).wait()

        def out_slice(l):
            return out_hbm.at[l, pl.ds(w * bpw, bpw)]

        def fire_write(l, b):
            pltpu.async_copy(bufs[b], out_slice(l), wsems[b])

        def wait_write(l, b):
            pltpu.make_async_copy(bufs[b], out_slice(l), wsems[b]).wait()

        for l in range(ahead):
            fire_gather(l, l % nbuf)

        def step(k, carry):
            for b in range(nbuf):
                l = k * nbuf + b
                wait_gather(l, b)
                fire_write(l, b)
                bn = (b + ahead) % nbuf

                @pl.when(l + ahead - nbuf >= 0)
                def _():
                    wait_write(l + ahead - nbuf, bn)

                @pl.when(l + ahead < hist)
                def _():
                    fire_gather(l + ahead, bn)
            return carry

        lax.fori_loop(0, hist // nbuf, step, 0)

        for l in range(hist - nbuf + ahead, hist):
            wait_write(l, l % nbuf)

    return gather_kernel


def kernel(ids, table):
    b, l = ids.shape
    out_t = _build_gather(b, l, table.shape[1])(ids.T.astype(jnp.int32), table)
    return out_t.transpose(1, 0, 2)


# speedup vs baseline: 10.6824x; 1.0083x over previous
import functools

import jax
import jax.numpy as jnp
from jax import lax
from jax.experimental import pallas as pl
from jax.experimental.pallas import tpu as pltpu
from jax.experimental.pallas import tpu_sc as plsc

NC = 2
NS = 16
NW = NC * NS


def _build_gather(batch: int, hist: int, emb_dim: int):
    assert batch % NW == 0
    bpw = batch // NW
    nbuf = 5
    ahead = 3
    assert hist % nbuf == 0

    mesh = plsc.VectorSubcoreMesh(core_axis_name="c", subcore_axis_name="s")

    @functools.partial(
        pl.kernel,
        out_type=jax.ShapeDtypeStruct((hist, batch, emb_dim), jnp.float32),
        mesh=mesh,
        compiler_params=pltpu.CompilerParams(use_tc_tiling_on_sc=True),
        scratch_types=[
            pltpu.VMEM((hist, bpw), jnp.int32),
            [pltpu.VMEM((bpw, emb_dim), jnp.float32) for _ in range(nbuf)],
            [pltpu.SemaphoreType.DMA for _ in range(nbuf)],
            [pltpu.SemaphoreType.DMA for _ in range(nbuf)],
        ],
    )
    def gather_kernel(ids_hbm, table_hbm, out_hbm, idx_v, bufs, gsems, wsems):
        w = lax.axis_index("s") * NC + lax.axis_index("c")
        pltpu.sync_copy(ids_hbm.at[:, pl.ds(w * bpw, bpw)], idx_v)

        def fire_gather(l, b):
            pltpu.async_copy(table_hbm.at[idx_v.at[l]], bufs[b], gsems[b])

        def wait_gather(l, b):
            pltpu.make_async_copy(table_hbm.at[idx_v.at[l]], bufs[b], gsems[b]).wait()

        def out_slice(l):
            return out_hbm.at[l, pl.ds(w * bpw, bpw)]

        def fire_write(l, b):
            pltpu.async_copy(bufs[b], out_slice(l), wsems[b])

        def wait_write(l, b):
            pltpu.make_async_copy(bufs[b], out_slice(l), wsems[b]).wait()

        for l in range(ahead):
            fire_gather(l, l % nbuf)

        def step(k, carry):
            for b in range(nbuf):
                l = k * nbuf + b
                wait_gather(l, b)
                fire_write(l, b)
                bn = (b + ahead) % nbuf

                @pl.when(l + ahead - nbuf >= 0)
                def _():
                    wait_write(l + ahead - nbuf, bn)

                @pl.when(l + ahead < hist)
                def _():
                    fire_gather(l + ahead, bn)
            return carry

        lax.fori_loop(0, hist // nbuf, step, 0)

        for l in range(hist - nbuf + ahead, hist):
            wait_write(l, l % nbuf)

    return gather_kernel


def kernel(ids, table):
    b, l = ids.shape
    out_t = _build_gather(b, l, table.shape[1])(ids.T.astype(jnp.int32), table)
    return out_t.transpose(1, 0, 2)


# speedup vs baseline: 10.7364x; 1.0051x over previous
import functools

import jax
import jax.numpy as jnp
from jax import lax
from jax.experimental import pallas as pl
from jax.experimental.pallas import tpu as pltpu
from jax.experimental.pallas import tpu_sc as plsc

NC = 2
NS = 16
NW = NC * NS


def _build_gather(batch: int, hist: int, emb_dim: int):
    assert batch % NW == 0
    bpw = batch // NW
    nbuf = 5
    ahead = 4
    assert hist % nbuf == 0

    mesh = plsc.VectorSubcoreMesh(core_axis_name="c", subcore_axis_name="s")

    @functools.partial(
        pl.kernel,
        out_type=jax.ShapeDtypeStruct((hist, batch, emb_dim), jnp.float32),
        mesh=mesh,
        compiler_params=pltpu.CompilerParams(use_tc_tiling_on_sc=True),
        scratch_types=[
            pltpu.VMEM((hist, bpw), jnp.int32),
            [pltpu.VMEM((bpw, emb_dim), jnp.float32) for _ in range(nbuf)],
            [pltpu.SemaphoreType.DMA for _ in range(nbuf)],
            [pltpu.SemaphoreType.DMA for _ in range(nbuf)],
        ],
    )
    def gather_kernel(ids_hbm, table_hbm, out_hbm, idx_v, bufs, gsems, wsems):
        w = lax.axis_index("s") * NC + lax.axis_index("c")
        pltpu.sync_copy(ids_hbm.at[:, pl.ds(w * bpw, bpw)], idx_v)

        def fire_gather(l, b):
            pltpu.async_copy(table_hbm.at[idx_v.at[l]], bufs[b], gsems[b])

        def wait_gather(l, b):
            pltpu.make_async_copy(table_hbm.at[idx_v.at[l]], bufs[b], gsems[b]).wait()

        def out_slice(l):
            return out_hbm.at[l, pl.ds(w * bpw, bpw)]

        def fire_write(l, b):
            pltpu.async_copy(bufs[b], out_slice(l), wsems[b])

        def wait_write(l, b):
            pltpu.make_async_copy(bufs[b], out_slice(l), wsems[b]).wait()

        for l in range(ahead):
            fire_gather(l, l % nbuf)

        def step(k, carry):
            for b in range(nbuf):
                l = k * nbuf + b
                wait_gather(l, b)
                fire_write(l, b)
                bn = (b + ahead) % nbuf

                @pl.when(l + ahead - nbuf >= 0)
                def _():
                    wait_write(l + ahead - nbuf, bn)

                @pl.when(l + ahead < hist)
                def _():
                    fire_gather(l + ahead, bn)
            return carry

        lax.fori_loop(0, hist // nbuf, step, 0)

        for l in range(hist - nbuf + ahead, hist):
            wait_write(l, l % nbuf)

    return gather_kernel


def kernel(ids, table):
    b, l = ids.shape
    out_t = _build_gather(b, l, table.shape[1])(ids.T.astype(jnp.int32), table)
    return out_t.transpose(1, 0, 2)
